# Initial kernel scaffold; baseline (speedup 1.0000x reference)
import jax
import jax.numpy as jnp
from jax.experimental import pallas as pl


def kernel(h_src, h_dst, edge_index, seed_score):
    raise NotImplementedError("write your pallas kernel here")


# speedup vs baseline: 2.2925x; 2.2925x over previous
import functools

import jax
import jax.numpy as jnp
from jax import lax
from jax.experimental import pallas as pl
from jax.experimental.pallas import tpu as pltpu
from jax.experimental.pallas import tpu_sc as plsc

D = 128
NC, NS, L = 2, 16, 16
NW = NC * NS
C = 80


def _sc_body(hsrc, hdst, sidx_hbm, didx_hbm, seed_hbm, out_hbm,
             sidx, didx, seed, srows, drows, outb, tmp, sem, e_per_w):
    wid = lax.axis_index("s") * NC + lax.axis_index("c")
    base0 = wid * e_per_w
    rows0 = lax.iota(jnp.int32, L)

    def chunk_body(g, carry):
        base = base0 + g * C
        pltpu.sync_copy(sidx_hbm.at[pl.ds(base, C)], sidx)
        pltpu.sync_copy(didx_hbm.at[pl.ds(base, C)], didx)
        pltpu.sync_copy(seed_hbm.at[pl.ds(base, C)], seed)
        cp_s = pltpu.async_copy(hsrc.at[sidx], srows, sem)
        cp_d = pltpu.async_copy(hdst.at[didx], drows, sem)
        cp_s.wait()
        cp_d.wait()
        for e16 in range(C // L):
            for l in range(L):
                e = e16 * L + l
                acc = srows[e, pl.ds(0, L)] * drows[e, pl.ds(0, L)]
                for k in range(1, D // L):
                    acc = acc + srows[e, pl.ds(k * L, L)] * drows[e, pl.ds(k * L, L)]
                tmp[pl.ds(l * L, L)] = acc
            acc2 = seed[pl.ds(e16 * L, L)]
            for l in range(L):
                acc2 = acc2 + plsc.load_gather(tmp, [rows0 * L + l])
            outb[pl.ds(e16 * L, L)] = acc2
        pltpu.sync_copy(outb, out_hbm.at[pl.ds(base, C)])
        return carry

    lax.fori_loop(0, e_per_w // C, chunk_body, 0)


def kernel(h_src, h_dst, edge_index, seed_score):
    E = seed_score.shape[0]
    assert E % (NW * C) == 0
    e_per_w = E // NW
    src = edge_index[0].astype(jnp.int32)
    dst = edge_index[1].astype(jnp.int32)

    mesh = plsc.VectorSubcoreMesh(core_axis_name="c", subcore_axis_name="s")
    body = functools.partial(_sc_body, e_per_w=e_per_w)
    run = pl.kernel(
        body,
        out_type=jax.ShapeDtypeStruct((E,), jnp.float32),
        mesh=mesh,
        scratch_types=[
            pltpu.VMEM((C,), jnp.int32),
            pltpu.VMEM((C,), jnp.int32),
            pltpu.VMEM((C,), jnp.float32),
            pltpu.VMEM((C, D), jnp.float32),
            pltpu.VMEM((C, D), jnp.float32),
            pltpu.VMEM((C,), jnp.float32),
            pltpu.VMEM((L * L,), jnp.float32),
            pltpu.SemaphoreType.DMA,
        ],
        compiler_params=pltpu.CompilerParams(needs_layout_passes=False),
    )
    return run(h_src, h_dst, src, dst, seed_score)


# speedup vs baseline: 3.7801x; 1.6489x over previous
import functools

import jax
import jax.numpy as jnp
from jax import lax
from jax.experimental import pallas as pl
from jax.experimental.pallas import tpu as pltpu
from jax.experimental.pallas import tpu_sc as plsc

D = 128
NC, NS, L = 2, 16, 16
NW = NC * NS
C = 80


def _sc_body(hsrc, hdst, sidx_hbm, didx_hbm, seed_hbm, out_hbm,
             sidx, didx, seed, out_all, srows0, drows0, srows1, drows1, tmp,
             sem_in, sem0, sem1, e_per_w):
    wid = lax.axis_index("s") * NC + lax.axis_index("c")
    base0 = wid * e_per_w
    n_chunks = e_per_w // C
    rows0 = lax.iota(jnp.int32, L)

    cps = [pltpu.async_copy(sidx_hbm.at[pl.ds(base0, e_per_w)], sidx, sem_in),
           pltpu.async_copy(didx_hbm.at[pl.ds(base0, e_per_w)], didx, sem_in),
           pltpu.async_copy(seed_hbm.at[pl.ds(base0, e_per_w)], seed, sem_in)]
    for cp in cps:
        cp.wait()

    def issue(g, srows, drows, sem):
        off = g * C
        pltpu.async_copy(hsrc.at[sidx.at[pl.ds(off, C)]], srows, sem)
        pltpu.async_copy(hdst.at[didx.at[pl.ds(off, C)]], drows, sem)

    def wait_bufs(g, srows, drows, sem):
        off = g * C
        pltpu.make_async_copy(hsrc.at[sidx.at[pl.ds(off, C)]], srows, sem).wait()
        pltpu.make_async_copy(hdst.at[didx.at[pl.ds(off, C)]], drows, sem).wait()

    def compute(g, srows, drows):
        off = g * C
        for e16 in range(C // L):
            for l in range(L):
                e = e16 * L + l
                acc = srows[e, pl.ds(0, L)] * drows[e, pl.ds(0, L)]
                for k in range(1, D // L):
                    acc = acc + srows[e, pl.ds(k * L, L)] * drows[e, pl.ds(k * L, L)]
                tmp[pl.ds(l * L, L)] = acc
            acc2 = seed[pl.ds(off + e16 * L, L)]
            for l in range(L):
                acc2 = acc2 + plsc.load_gather(tmp, [rows0 * L + l])
            out_all[pl.ds(off + e16 * L, L)] = acc2

    def step(g, srows, drows, sem):
        wait_bufs(g, srows, drows, sem)
        compute(g, srows, drows)

        @pl.when(g + 2 < n_chunks)
        def _():
            issue(g + 2, srows, drows, sem)

    issue(0, srows0, drows0, sem0)
    issue(1, srows1, drows1, sem1)
    step(0, srows0, drows0, sem0)

    def pair_body(i, carry):
        step(2 * i + 1, srows1, drows1, sem1)
        step(2 * i + 2, srows0, drows0, sem0)
        return carry

    lax.fori_loop(0, (n_chunks - 1) // 2, pair_body, 0)

    pltpu.sync_copy(out_all, out_hbm.at[pl.ds(base0, e_per_w)])


def kernel(h_src, h_dst, edge_index, seed_score):
    E = seed_score.shape[0]
    assert E % (NW * C) == 0 and (E // (NW * C)) % 2 == 1
    e_per_w = E // NW
    src = edge_index[0].astype(jnp.int32)
    dst = edge_index[1].astype(jnp.int32)

    mesh = plsc.VectorSubcoreMesh(core_axis_name="c", subcore_axis_name="s")
    body = functools.partial(_sc_body, e_per_w=e_per_w)
    run = pl.kernel(
        body,
        out_type=jax.ShapeDtypeStruct((E,), jnp.float32),
        mesh=mesh,
        scratch_types=[
            pltpu.VMEM((e_per_w,), jnp.int32),
            pltpu.VMEM((e_per_w,), jnp.int32),
            pltpu.VMEM((e_per_w,), jnp.float32),
            pltpu.VMEM((e_per_w,), jnp.float32),
            pltpu.VMEM((C, D), jnp.float32),
            pltpu.VMEM((C, D), jnp.float32),
            pltpu.VMEM((C, D), jnp.float32),
            pltpu.VMEM((C, D), jnp.float32),
            pltpu.VMEM((L * L,), jnp.float32),
            pltpu.SemaphoreType.DMA,
            pltpu.SemaphoreType.DMA,
            pltpu.SemaphoreType.DMA,
        ],
        compiler_params=pltpu.CompilerParams(needs_layout_passes=False),
    )
    return run(h_src, h_dst, src, dst, seed_score)
